# baseline (speedup 1.0000x reference)
import functools
import math

import jax
import jax.numpy as jnp
from jax import lax
from jax.experimental import pallas as pl
from jax.experimental.pallas import tpu as pltpu
from jax.experimental.pallas import tpu_sc as plsc

N = 50000
E = 800000
D_IN = 512
H0, H1 = 256, 128
HID, PROJ = 64, 64
EPS = 0.001
_BNS = 1.0 / math.sqrt(1.0 + EPS)

BR = 1000
NB = N // BR

NTILE = 16
CHUNK = 128
NCH = E // CHUNK
CPT = 391
XTILES = NCH - NTILE * (CPT - 1)
NBUF = 5
SEG = 30
NSEG = (CPT - 1) // SEG
RPT = 3136
RLAST = N - 15 * RPT

_f32 = jnp.float32


def _elu(v):
    return jnp.where(v > 0, v, jnp.exp(jnp.minimum(v, 0.0)) - 1.0)


_bf16 = jnp.bfloat16


def _enc_body(x_ref, w0_ref, b0_ref, g0_ref, t0_ref, w1_ref, b1_ref, g1_ref,
              t1_ref, o_ref):
    h = jnp.dot(x_ref[...].astype(_bf16), w0_ref[...].astype(_bf16),
                preferred_element_type=_f32)
    h = (h + b0_ref[...]) * (g0_ref[...] * _BNS) + t0_ref[...]
    h = _elu(h)
    h = jnp.dot(h.astype(_bf16), w1_ref[...].astype(_bf16),
                preferred_element_type=_f32)
    h = (h + b1_ref[...]) * (g1_ref[...] * _BNS) + t1_ref[...]
    o_ref[...] = _elu(h)


def _row_spec(w):
    return pl.BlockSpec((BR, w), lambda i: (i, 0))


def _full_spec(shape):
    return pl.BlockSpec(shape, lambda i: (0,) * len(shape))


def _encoder(x, w0, b0, g0, t0, w1, b1, g1, t1):
    return pl.pallas_call(
        _enc_body,
        grid=(NB,),
        in_specs=[_row_spec(D_IN), _full_spec((D_IN, H0)), _full_spec((1, H0)),
                  _full_spec((1, H0)), _full_spec((1, H0)),
                  _full_spec((H0, H1)), _full_spec((1, H1)),
                  _full_spec((1, H1)), _full_spec((1, H1))],
        out_specs=_row_spec(H1),
        out_shape=jax.ShapeDtypeStruct((N, H1), _f32),
    )(x, w0, b0.reshape(1, H0), g0.reshape(1, H0), t0.reshape(1, H0),
      w1, b1.reshape(1, H1), g1.reshape(1, H1), t1.reshape(1, H1))


def _prep_body(z_ref, w_ref, cnt_ref, ylo_ref, yhi_ref):
    xw = jnp.dot(z_ref[...], w_ref[...], preferred_element_type=_f32)
    dinv = lax.rsqrt(cnt_ref[:, 0:1] + 1.0)
    y = xw * dinv
    ylo_ref[...] = y[:, :32]
    yhi_ref[...] = y[:, 32:]


def _gcn_prep(z, w, cnt):
    return pl.pallas_call(
        _prep_body,
        grid=(NB,),
        in_specs=[_row_spec(H1), _full_spec((H1, HID)), _row_spec(8)],
        out_specs=[_row_spec(32), _row_spec(32)],
        out_shape=[jax.ShapeDtypeStruct((N, 32), _f32)] * 2,
    )(z, w, cnt)


def _mid_body(plo_ref, phi_ref, cnt_ref, b_ref, a_ref, w_ref,
              ylo_ref, yhi_ref):
    a = a_ref[0, 0]
    dinv = lax.rsqrt(cnt_ref[:, 0:1] + 1.0)
    p = jnp.concatenate([plo_ref[...], phi_ref[...]], axis=1)
    h = p * dinv + b_ref[...]
    h = jnp.where(h >= 0, h, a * h)
    xw = jnp.dot(h, w_ref[...], preferred_element_type=_f32)
    y = xw * dinv
    ylo_ref[...] = y[:, :32]
    yhi_ref[...] = y[:, 32:]


def _gcn_mid(plo, phi, cnt, b, a, w):
    return pl.pallas_call(
        _mid_body,
        grid=(NB,),
        in_specs=[_row_spec(32), _row_spec(32), _row_spec(8),
                  _full_spec((1, HID)), _full_spec((1, 1)),
                  _full_spec((HID, PROJ))],
        out_specs=[_row_spec(32), _row_spec(32)],
        out_shape=[jax.ShapeDtypeStruct((N, 32), _f32)] * 2,
    )(plo, phi, cnt, b.reshape(1, HID), a.reshape(1, 1), w)


def _finproj_body(plo_ref, phi_ref, cnt_ref, cb_ref, a_ref,
                  f1w_ref, f1b_ref, f2w_ref, f2b_ref,
                  h_ref, e_ref):
    a = a_ref[0, 0]
    dinv = lax.rsqrt(cnt_ref[:, 0:1] + 1.0)
    p = jnp.concatenate([plo_ref[...], phi_ref[...]], axis=1)
    e = p * dinv + cb_ref[...]
    e = jnp.where(e >= 0, e, a * e)
    e_ref[...] = e
    t = _elu(jnp.dot(e, f1w_ref[...], preferred_element_type=_f32)
             + f1b_ref[...])
    h_ref[...] = jnp.dot(t, f2w_ref[...],
                         preferred_element_type=_f32) + f2b_ref[...]


def _finproj(plo, phi, cnt, cb, a, f1w, f1b, f2w, f2b):
    return pl.pallas_call(
        _finproj_body,
        grid=(NB,),
        in_specs=[_row_spec(32), _row_spec(32), _row_spec(8),
                  _full_spec((1, PROJ)), _full_spec((1, 1)),
                  _full_spec((PROJ, PROJ)), _full_spec((1, PROJ)),
                  _full_spec((PROJ, PROJ)), _full_spec((1, PROJ))],
        out_specs=[_row_spec(PROJ), _row_spec(PROJ)],
        out_shape=[jax.ShapeDtypeStruct((N, PROJ), _f32),
                   jax.ShapeDtypeStruct((N, PROJ), _f32)],
    )(plo, phi, cnt, cb.reshape(1, PROJ), a.reshape(1, 1),
      f1w, f1b.reshape(1, PROJ), f2w, f2b.reshape(1, PROJ))


def _att_body(e1_ref, e2_ref, aw_ref, dw_ref, db_ref, dg_ref, dt_ref,
              emb_ref, dz_ref):
    e1 = e1_ref[...]
    e2 = e2_ref[...]
    w1 = jnp.dot(e1, aw_ref[...], preferred_element_type=_f32)
    w2 = jnp.dot(e2, aw_ref[...], preferred_element_type=_f32)
    m = jnp.maximum(w1, w2)
    x1 = jnp.exp(w1 - m)
    x2 = jnp.exp(w2 - m)
    b1 = x1 / (x1 + x2)
    emb = b1 * e1 + (1.0 - b1) * e2
    emb_ref[...] = emb

    d = jnp.dot(emb, dw_ref[...], preferred_element_type=_f32)
    d = (d + db_ref[...]) * (dg_ref[...] * _BNS) + dt_ref[...]
    dz_ref[...] = 1.0 / (1.0 + jnp.exp(-d))


def _att_dec(e1, e2, aw, dw, db, dg, dt):
    return pl.pallas_call(
        _att_body,
        grid=(NB,),
        in_specs=[_row_spec(PROJ), _row_spec(PROJ),
                  _full_spec((PROJ, 1)),
                  _full_spec((PROJ, D_IN)), _full_spec((1, D_IN)),
                  _full_spec((1, D_IN)), _full_spec((1, D_IN))],
        out_specs=[_row_spec(PROJ), _row_spec(D_IN)],
        out_shape=[jax.ShapeDtypeStruct((N, PROJ), _f32),
                   jax.ShapeDtypeStruct((N, D_IN), _f32)],
    )(e1, e2, aw,
      dw, db.reshape(1, D_IN), dg.reshape(1, D_IN), dt.reshape(1, D_IN))


_MESH = plsc.VectorSubcoreMesh(core_axis_name="c", subcore_axis_name="s")
_SC_PARAMS = pltpu.CompilerParams(use_tc_tiling_on_sc=False)


def _node_rows(sid, do_full, do_last):
    row0 = sid * RPT

    @pl.when(sid < NTILE - 1)
    def _():
        do_full(row0)

    @pl.when(sid == NTILE - 1)
    def _():
        do_last(row0)


@functools.partial(
    pl.kernel,
    out_type=[jax.ShapeDtypeStruct((N, 32), _f32),
              jax.ShapeDtypeStruct((N, 32), _f32)],
    mesh=_MESH,
    scratch_types=[pltpu.VMEM_SHARED((N, 32), _f32),
                   pltpu.VMEM((SEG, CHUNK), jnp.int32),
                   pltpu.VMEM((SEG, CHUNK), jnp.int32),
                   [pltpu.VMEM((CHUNK, 32), _f32) for _ in range(NBUF)],
                   [pltpu.SemaphoreType.DMA for _ in range(NBUF)],
                   [pltpu.SemaphoreType.DMA for _ in range(NBUF)]],
    compiler_params=_SC_PARAMS,
)
def _edge_pass(ylo, yhi, src2d, dst2d, plo, phi,
               acc, sidx, didx, rows, sems, sems2):
    cid = lax.axis_index("c")
    sid = lax.axis_index("s")

    cbase = (CPT - 1) * sid + jnp.minimum(sid, XTILES)
    has_extra = sid < XTILES

    def run(y_hbm, out_hbm):
        _node_rows(
            sid,
            lambda r0: pltpu.sync_copy(y_hbm.at[pl.ds(r0, RPT)],
                                       acc.at[pl.ds(r0, RPT)]),
            lambda r0: pltpu.sync_copy(y_hbm.at[pl.ds(r0, RLAST)],
                                       acc.at[pl.ds(r0, RLAST)]),
        )
        plsc.subcore_barrier()

        def start(c, b):
            pltpu.async_copy(y_hbm.at[sidx.at[c]], rows[b], sems[b])

        def wait(b):
            pltpu.make_async_copy(y_hbm.at[pl.ds(0, CHUNK)], rows[b],
                                  sems[b]).wait()

        def scat(c, b):
            pltpu.async_copy(rows[b], acc.at[didx.at[c]], sems2[b], add=True)

        def wait_scat(b):
            pltpu.make_async_copy(y_hbm.at[pl.ds(0, CHUNK)], rows[b],
                                  sems2[b]).wait()

        def seg_body(s, _):
            cb = cbase + s * SEG
            pltpu.sync_copy(src2d.at[pl.ds(cb, SEG)], sidx)
            pltpu.sync_copy(dst2d.at[pl.ds(cb, SEG)], didx)

            for b in range(NBUF):
                start(b, b)

            def group(g, _):
                c0 = g * NBUF
                for b in range(NBUF):
                    wait(b)
                    scat(c0 + b, b)
                    nxt = c0 + b + NBUF

                    @pl.when(nxt < SEG)
                    def _():
                        wait_scat(b)
                        start(nxt, b)
                return 0

            lax.fori_loop(0, SEG // NBUF, group, 0)

            for b in range(NBUF):
                wait_scat(b)
            return 0

        lax.fori_loop(0, NSEG, seg_body, 0)

        @pl.when(has_extra)
        def _():
            pltpu.sync_copy(src2d.at[pl.ds(cbase + CPT - 1, 1)],
                            sidx.at[pl.ds(0, 1)])
            pltpu.sync_copy(dst2d.at[pl.ds(cbase + CPT - 1, 1)],
                            didx.at[pl.ds(0, 1)])
            start(0, 0)
            wait(0)
            scat(0, 0)
            wait_scat(0)

        plsc.subcore_barrier()

        _node_rows(
            sid,
            lambda r0: pltpu.sync_copy(acc.at[pl.ds(r0, RPT)],
                                       out_hbm.at[pl.ds(r0, RPT)]),
            lambda r0: pltpu.sync_copy(acc.at[pl.ds(r0, RLAST)],
                                       out_hbm.at[pl.ds(r0, RLAST)]),
        )

    @pl.when(cid == 0)
    def _():
        run(ylo, plo)

    @pl.when(cid == 1)
    def _():
        run(yhi, phi)


@functools.partial(
    pl.kernel,
    out_type=[jax.ShapeDtypeStruct((N, 8), _f32),
              jax.ShapeDtypeStruct((N, 8), _f32)],
    mesh=_MESH,
    scratch_types=[pltpu.VMEM_SHARED((N, 8), _f32),
                   pltpu.VMEM((CPT, CHUNK), jnp.int32),
                   pltpu.VMEM((CHUNK, 8), _f32)],
    compiler_params=_SC_PARAMS,
)
def _degrees_sc(dst2d1, dst2d2, zeros, ones, cnt1, cnt2, acc, didx, ones_v):
    cid = lax.axis_index("c")
    sid = lax.axis_index("s")

    pltpu.sync_copy(ones, ones_v)

    cbase = (CPT - 1) * sid + jnp.minimum(sid, XTILES)
    has_extra = sid < XTILES

    def run(dst2d, out_hbm):
        pltpu.sync_copy(dst2d.at[pl.ds(cbase, CPT - 1)],
                        didx.at[pl.ds(0, CPT - 1)])

        @pl.when(has_extra)
        def _():
            pltpu.sync_copy(dst2d.at[pl.ds(cbase + CPT - 1, 1)],
                            didx.at[pl.ds(CPT - 1, 1)])

        _node_rows(
            sid,
            lambda r0: pltpu.sync_copy(zeros.at[pl.ds(r0, RPT)],
                                       acc.at[pl.ds(r0, RPT)]),
            lambda r0: pltpu.sync_copy(zeros.at[pl.ds(r0, RLAST)],
                                       acc.at[pl.ds(r0, RLAST)]),
        )
        plsc.subcore_barrier()

        def chunk(j, _):
            pltpu.sync_copy(ones_v, acc.at[didx.at[j]], add=True)
            return 0

        lax.fori_loop(0, CPT - 1, chunk, 0)

        @pl.when(has_extra)
        def _():
            pltpu.sync_copy(ones_v, acc.at[didx.at[CPT - 1]], add=True)

        plsc.subcore_barrier()

        _node_rows(
            sid,
            lambda r0: pltpu.sync_copy(acc.at[pl.ds(r0, RPT)],
                                       out_hbm.at[pl.ds(r0, RPT)]),
            lambda r0: pltpu.sync_copy(acc.at[pl.ds(r0, RLAST)],
                                       out_hbm.at[pl.ds(r0, RLAST)]),
        )

    @pl.when(cid == 0)
    def _():
        run(dst2d1, cnt1)

    @pl.when(cid == 1)
    def _():
        run(dst2d2, cnt2)


def kernel(x, edge_index_1, edge_index_2,
           enc_W0, enc_b0, enc_g0, enc_bt0,
           enc_W1, enc_b1, enc_g1, enc_bt1,
           conv_W0, conv_b0, conv_W1, conv_b1, prelu_a,
           fc1_W, fc1_b, fc2_W, fc2_b, att_W,
           dec_W, dec_b, dec_g, dec_bt):
    z = _encoder(x, enc_W0, enc_b0, enc_g0, enc_bt0,
                 enc_W1, enc_b1, enc_g1, enc_bt1)

    src1 = edge_index_1[0].reshape(NCH, CHUNK)
    dst1 = edge_index_1[1].reshape(NCH, CHUNK)
    src2 = edge_index_2[0].reshape(NCH, CHUNK)
    dst2 = edge_index_2[1].reshape(NCH, CHUNK)

    zeros8 = jnp.zeros((N, 8), _f32)
    ones8 = jnp.ones((CHUNK, 8), _f32)
    cnt1, cnt2 = _degrees_sc(dst1, dst2, zeros8, ones8)

    def branch(src, dst, cnt):
        ylo, yhi = _gcn_prep(z, conv_W0, cnt)
        plo, phi = _edge_pass(ylo, yhi, src, dst)
        yblo, ybhi = _gcn_mid(plo, phi, cnt, conv_b0, prelu_a, conv_W1)
        qlo, qhi = _edge_pass(yblo, ybhi, src, dst)
        return _finproj(qlo, qhi, cnt, conv_b1, prelu_a,
                        fc1_W, fc1_b, fc2_W, fc2_b)

    h1, emb1 = branch(src1, dst1, cnt1)
    h2, emb2 = branch(src2, dst2, cnt2)

    emb, de_z = _att_dec(emb1, emb2, att_W, dec_W, dec_b, dec_g, dec_bt)
    return (h1, h2, emb, de_z)


# speedup vs baseline: 1.0209x; 1.0209x over previous
import functools
import math

import jax
import jax.numpy as jnp
from jax import lax
from jax.experimental import pallas as pl
from jax.experimental.pallas import tpu as pltpu
from jax.experimental.pallas import tpu_sc as plsc

N = 50000
E = 800000
D_IN = 512
H0, H1 = 256, 128
HID, PROJ = 64, 64
EPS = 0.001
_BNS = 1.0 / math.sqrt(1.0 + EPS)

BR = 1000
NB = N // BR

NTILE = 16
CHUNK = 128
NCH = E // CHUNK
CPT = 391
XTILES = NCH - NTILE * (CPT - 1)
NBUF = 5
SEG = 30
NSEG = (CPT - 1) // SEG
RPT = 3136
RLAST = N - 15 * RPT

_f32 = jnp.float32


def _elu(v):
    return jnp.where(v > 0, v, jnp.exp(jnp.minimum(v, 0.0)) - 1.0)


_bf16 = jnp.bfloat16


def _enc_body(x_ref, w0_ref, b0_ref, g0_ref, t0_ref, w1_ref, b1_ref, g1_ref,
              t1_ref, o_ref):
    h = jnp.dot(x_ref[...].astype(_bf16), w0_ref[...].astype(_bf16),
                preferred_element_type=_f32)
    h = (h + b0_ref[...]) * (g0_ref[...] * _BNS) + t0_ref[...]
    h = _elu(h)
    h = jnp.dot(h.astype(_bf16), w1_ref[...].astype(_bf16),
                preferred_element_type=_f32)
    h = (h + b1_ref[...]) * (g1_ref[...] * _BNS) + t1_ref[...]
    o_ref[...] = _elu(h)


def _row_spec(w):
    return pl.BlockSpec((BR, w), lambda i: (i, 0))


def _full_spec(shape):
    return pl.BlockSpec(shape, lambda i: (0,) * len(shape))


def _encoder(x, w0, b0, g0, t0, w1, b1, g1, t1):
    return pl.pallas_call(
        _enc_body,
        grid=(NB,),
        in_specs=[_row_spec(D_IN), _full_spec((D_IN, H0)), _full_spec((1, H0)),
                  _full_spec((1, H0)), _full_spec((1, H0)),
                  _full_spec((H0, H1)), _full_spec((1, H1)),
                  _full_spec((1, H1)), _full_spec((1, H1))],
        out_specs=_row_spec(H1),
        out_shape=jax.ShapeDtypeStruct((N, H1), _f32),
    )(x, w0, b0.reshape(1, H0), g0.reshape(1, H0), t0.reshape(1, H0),
      w1, b1.reshape(1, H1), g1.reshape(1, H1), t1.reshape(1, H1))


def _prep_body(z_ref, w_ref, cnt_ref, ylo_ref, yhi_ref):
    xw = jnp.dot(z_ref[...], w_ref[...], preferred_element_type=_f32)
    dinv = lax.rsqrt(cnt_ref[:, 0:1] + 1.0)
    y = xw * dinv
    ylo_ref[...] = y[:, :32]
    yhi_ref[...] = y[:, 32:]


def _gcn_prep(z, w, cnt):
    return pl.pallas_call(
        _prep_body,
        grid=(NB,),
        in_specs=[_row_spec(H1), _full_spec((H1, HID)), _row_spec(8)],
        out_specs=[_row_spec(32), _row_spec(32)],
        out_shape=[jax.ShapeDtypeStruct((N, 32), _f32)] * 2,
    )(z, w, cnt)


def _mid_body(plo_ref, phi_ref, cnt_ref, b_ref, a_ref, w_ref,
              ylo_ref, yhi_ref):
    a = a_ref[0, 0]
    dinv = lax.rsqrt(cnt_ref[:, 0:1] + 1.0)
    p = jnp.concatenate([plo_ref[...], phi_ref[...]], axis=1)
    h = p * dinv + b_ref[...]
    h = jnp.where(h >= 0, h, a * h)
    xw = jnp.dot(h, w_ref[...], preferred_element_type=_f32)
    y = xw * dinv
    ylo_ref[...] = y[:, :32]
    yhi_ref[...] = y[:, 32:]


def _gcn_mid(plo, phi, cnt, b, a, w):
    return pl.pallas_call(
        _mid_body,
        grid=(NB,),
        in_specs=[_row_spec(32), _row_spec(32), _row_spec(8),
                  _full_spec((1, HID)), _full_spec((1, 1)),
                  _full_spec((HID, PROJ))],
        out_specs=[_row_spec(32), _row_spec(32)],
        out_shape=[jax.ShapeDtypeStruct((N, 32), _f32)] * 2,
    )(plo, phi, cnt, b.reshape(1, HID), a.reshape(1, 1), w)


def _head_body(p1lo_ref, p1hi_ref, p2lo_ref, p2hi_ref, cnt1_ref, cnt2_ref,
               cb_ref, a_ref,
               f1w_ref, f1b_ref, f2w_ref, f2b_ref, aw_ref,
               dw_ref, db_ref, dg_ref, dt_ref,
               h1_ref, h2_ref, emb_ref, dz_ref):
    a = a_ref[0, 0]

    def fin(plo_ref, phi_ref, cnt_ref):
        dinv = lax.rsqrt(cnt_ref[:, 0:1] + 1.0)
        p = jnp.concatenate([plo_ref[...], phi_ref[...]], axis=1)
        h = p * dinv + cb_ref[...]
        return jnp.where(h >= 0, h, a * h)

    e1 = fin(p1lo_ref, p1hi_ref, cnt1_ref)
    e2 = fin(p2lo_ref, p2hi_ref, cnt2_ref)

    def proj(e):
        t = _elu(jnp.dot(e, f1w_ref[...], preferred_element_type=_f32)
                 + f1b_ref[...])
        return jnp.dot(t, f2w_ref[...], preferred_element_type=_f32) + f2b_ref[...]

    h1_ref[...] = proj(e1)
    h2_ref[...] = proj(e2)

    w1 = jnp.dot(e1, aw_ref[...], preferred_element_type=_f32)
    w2 = jnp.dot(e2, aw_ref[...], preferred_element_type=_f32)
    m = jnp.maximum(w1, w2)
    x1 = jnp.exp(w1 - m)
    x2 = jnp.exp(w2 - m)
    b1 = x1 / (x1 + x2)
    emb = b1 * e1 + (1.0 - b1) * e2
    emb_ref[...] = emb

    d = jnp.dot(emb, dw_ref[...], preferred_element_type=_f32)
    d = (d + db_ref[...]) * (dg_ref[...] * _BNS) + dt_ref[...]
    dz_ref[...] = 1.0 / (1.0 + jnp.exp(-d))


def _head(p1lo, p1hi, p2lo, p2hi, cnt1, cnt2, cb, a,
          f1w, f1b, f2w, f2b, aw, dw, db, dg, dt):
    return pl.pallas_call(
        _head_body,
        grid=(NB,),
        in_specs=[_row_spec(32), _row_spec(32), _row_spec(32), _row_spec(32),
                  _row_spec(8), _row_spec(8),
                  _full_spec((1, PROJ)), _full_spec((1, 1)),
                  _full_spec((PROJ, PROJ)), _full_spec((1, PROJ)),
                  _full_spec((PROJ, PROJ)), _full_spec((1, PROJ)),
                  _full_spec((PROJ, 1)),
                  _full_spec((PROJ, D_IN)), _full_spec((1, D_IN)),
                  _full_spec((1, D_IN)), _full_spec((1, D_IN))],
        out_specs=[_row_spec(PROJ), _row_spec(PROJ), _row_spec(PROJ),
                   _row_spec(D_IN)],
        out_shape=[jax.ShapeDtypeStruct((N, PROJ), _f32),
                   jax.ShapeDtypeStruct((N, PROJ), _f32),
                   jax.ShapeDtypeStruct((N, PROJ), _f32),
                   jax.ShapeDtypeStruct((N, D_IN), _f32)],
    )(p1lo, p1hi, p2lo, p2hi, cnt1, cnt2,
      cb.reshape(1, PROJ), a.reshape(1, 1),
      f1w, f1b.reshape(1, PROJ), f2w, f2b.reshape(1, PROJ), aw,
      dw, db.reshape(1, D_IN), dg.reshape(1, D_IN), dt.reshape(1, D_IN))


_MESH = plsc.VectorSubcoreMesh(core_axis_name="c", subcore_axis_name="s")
_SC_PARAMS = pltpu.CompilerParams(use_tc_tiling_on_sc=False)


def _node_rows(sid, do_full, do_last):
    row0 = sid * RPT

    @pl.when(sid < NTILE - 1)
    def _():
        do_full(row0)

    @pl.when(sid == NTILE - 1)
    def _():
        do_last(row0)


@functools.partial(
    pl.kernel,
    out_type=[jax.ShapeDtypeStruct((N, 32), _f32),
              jax.ShapeDtypeStruct((N, 32), _f32)],
    mesh=_MESH,
    scratch_types=[pltpu.VMEM_SHARED((N, 32), _f32),
                   pltpu.VMEM((SEG, CHUNK), jnp.int32),
                   pltpu.VMEM((SEG, CHUNK), jnp.int32),
                   [pltpu.VMEM((CHUNK, 32), _f32) for _ in range(NBUF)],
                   [pltpu.SemaphoreType.DMA for _ in range(NBUF)],
                   [pltpu.SemaphoreType.DMA for _ in range(NBUF)]],
    compiler_params=_SC_PARAMS,
)
def _edge_pass(ylo, yhi, src2d, dst2d, plo, phi,
               acc, sidx, didx, rows, sems, sems2):
    cid = lax.axis_index("c")
    sid = lax.axis_index("s")

    cbase = (CPT - 1) * sid + jnp.minimum(sid, XTILES)
    has_extra = sid < XTILES

    def run(y_hbm, out_hbm):
        _node_rows(
            sid,
            lambda r0: pltpu.sync_copy(y_hbm.at[pl.ds(r0, RPT)],
                                       acc.at[pl.ds(r0, RPT)]),
            lambda r0: pltpu.sync_copy(y_hbm.at[pl.ds(r0, RLAST)],
                                       acc.at[pl.ds(r0, RLAST)]),
        )
        plsc.subcore_barrier()

        def start(c, b):
            pltpu.async_copy(y_hbm.at[sidx.at[c]], rows[b], sems[b])

        def wait(b):
            pltpu.make_async_copy(y_hbm.at[pl.ds(0, CHUNK)], rows[b],
                                  sems[b]).wait()

        def scat(c, b):
            pltpu.async_copy(rows[b], acc.at[didx.at[c]], sems2[b], add=True)

        def wait_scat(b):
            pltpu.make_async_copy(y_hbm.at[pl.ds(0, CHUNK)], rows[b],
                                  sems2[b]).wait()

        def seg_body(s, _):
            cb = cbase + s * SEG
            pltpu.sync_copy(src2d.at[pl.ds(cb, SEG)], sidx)
            pltpu.sync_copy(dst2d.at[pl.ds(cb, SEG)], didx)

            for b in range(NBUF):
                start(b, b)

            def group(g, _):
                c0 = g * NBUF
                for b in range(NBUF):
                    wait(b)
                    scat(c0 + b, b)
                    nxt = c0 + b + NBUF

                    @pl.when(nxt < SEG)
                    def _():
                        wait_scat(b)
                        start(nxt, b)
                return 0

            lax.fori_loop(0, SEG // NBUF, group, 0)

            for b in range(NBUF):
                wait_scat(b)
            return 0

        lax.fori_loop(0, NSEG, seg_body, 0)

        @pl.when(has_extra)
        def _():
            pltpu.sync_copy(src2d.at[pl.ds(cbase + CPT - 1, 1)],
                            sidx.at[pl.ds(0, 1)])
            pltpu.sync_copy(dst2d.at[pl.ds(cbase + CPT - 1, 1)],
                            didx.at[pl.ds(0, 1)])
            start(0, 0)
            wait(0)
            scat(0, 0)
            wait_scat(0)

        plsc.subcore_barrier()

        _node_rows(
            sid,
            lambda r0: pltpu.sync_copy(acc.at[pl.ds(r0, RPT)],
                                       out_hbm.at[pl.ds(r0, RPT)]),
            lambda r0: pltpu.sync_copy(acc.at[pl.ds(r0, RLAST)],
                                       out_hbm.at[pl.ds(r0, RLAST)]),
        )

    @pl.when(cid == 0)
    def _():
        run(ylo, plo)

    @pl.when(cid == 1)
    def _():
        run(yhi, phi)


@functools.partial(
    pl.kernel,
    out_type=[jax.ShapeDtypeStruct((N, 8), _f32),
              jax.ShapeDtypeStruct((N, 8), _f32)],
    mesh=_MESH,
    scratch_types=[pltpu.VMEM_SHARED((N, 8), _f32),
                   pltpu.VMEM((CPT, CHUNK), jnp.int32),
                   pltpu.VMEM((CHUNK, 8), _f32)],
    compiler_params=_SC_PARAMS,
)
def _degrees_sc(dst2d1, dst2d2, zeros, ones, cnt1, cnt2, acc, didx, ones_v):
    cid = lax.axis_index("c")
    sid = lax.axis_index("s")

    pltpu.sync_copy(ones, ones_v)

    cbase = (CPT - 1) * sid + jnp.minimum(sid, XTILES)
    has_extra = sid < XTILES

    def run(dst2d, out_hbm):
        pltpu.sync_copy(dst2d.at[pl.ds(cbase, CPT - 1)],
                        didx.at[pl.ds(0, CPT - 1)])

        @pl.when(has_extra)
        def _():
            pltpu.sync_copy(dst2d.at[pl.ds(cbase + CPT - 1, 1)],
                            didx.at[pl.ds(CPT - 1, 1)])

        _node_rows(
            sid,
            lambda r0: pltpu.sync_copy(zeros.at[pl.ds(r0, RPT)],
                                       acc.at[pl.ds(r0, RPT)]),
            lambda r0: pltpu.sync_copy(zeros.at[pl.ds(r0, RLAST)],
                                       acc.at[pl.ds(r0, RLAST)]),
        )
        plsc.subcore_barrier()

        def chunk(j, _):
            pltpu.sync_copy(ones_v, acc.at[didx.at[j]], add=True)
            return 0

        lax.fori_loop(0, CPT - 1, chunk, 0)

        @pl.when(has_extra)
        def _():
            pltpu.sync_copy(ones_v, acc.at[didx.at[CPT - 1]], add=True)

        plsc.subcore_barrier()

        _node_rows(
            sid,
            lambda r0: pltpu.sync_copy(acc.at[pl.ds(r0, RPT)],
                                       out_hbm.at[pl.ds(r0, RPT)]),
            lambda r0: pltpu.sync_copy(acc.at[pl.ds(r0, RLAST)],
                                       out_hbm.at[pl.ds(r0, RLAST)]),
        )

    @pl.when(cid == 0)
    def _():
        run(dst2d1, cnt1)

    @pl.when(cid == 1)
    def _():
        run(dst2d2, cnt2)


def kernel(x, edge_index_1, edge_index_2,
           enc_W0, enc_b0, enc_g0, enc_bt0,
           enc_W1, enc_b1, enc_g1, enc_bt1,
           conv_W0, conv_b0, conv_W1, conv_b1, prelu_a,
           fc1_W, fc1_b, fc2_W, fc2_b, att_W,
           dec_W, dec_b, dec_g, dec_bt):
    z = _encoder(x, enc_W0, enc_b0, enc_g0, enc_bt0,
                 enc_W1, enc_b1, enc_g1, enc_bt1)

    src1 = edge_index_1[0].reshape(NCH, CHUNK)
    dst1 = edge_index_1[1].reshape(NCH, CHUNK)
    src2 = edge_index_2[0].reshape(NCH, CHUNK)
    dst2 = edge_index_2[1].reshape(NCH, CHUNK)

    zeros8 = jnp.zeros((N, 8), _f32)
    ones8 = jnp.ones((CHUNK, 8), _f32)
    cnt1, cnt2 = _degrees_sc(dst1, dst2, zeros8, ones8)

    def branch(src, dst, cnt):
        ylo, yhi = _gcn_prep(z, conv_W0, cnt)
        plo, phi = _edge_pass(ylo, yhi, src, dst)
        yblo, ybhi = _gcn_mid(plo, phi, cnt, conv_b0, prelu_a, conv_W1)
        return _edge_pass(yblo, ybhi, src, dst)

    q1lo, q1hi = branch(src1, dst1, cnt1)
    q2lo, q2hi = branch(src2, dst2, cnt2)

    return _head(q1lo, q1hi, q2lo, q2hi, cnt1, cnt2, conv_b1, prelu_a,
                 fc1_W, fc1_b, fc2_W, fc2_b, att_W,
                 dec_W, dec_b, dec_g, dec_bt)


# speedup vs baseline: 1.0212x; 1.0004x over previous
import functools
import math

import jax
import jax.numpy as jnp
from jax import lax
from jax.experimental import pallas as pl
from jax.experimental.pallas import tpu as pltpu
from jax.experimental.pallas import tpu_sc as plsc

N = 50000
E = 800000
D_IN = 512
H0, H1 = 256, 128
HID, PROJ = 64, 64
EPS = 0.001
_BNS = 1.0 / math.sqrt(1.0 + EPS)

BR = 1000
NB = N // BR

NTILE = 16
CHUNK = 128
NCH = E // CHUNK
CPT = 391
XTILES = NCH - NTILE * (CPT - 1)
NBUF = 5
SEG = 30
NSEG = (CPT - 1) // SEG
RPT = 3136
RLAST = N - 15 * RPT

_f32 = jnp.float32


def _elu(v):
    return jnp.where(v > 0, v, jnp.exp(jnp.minimum(v, 0.0)) - 1.0)


_bf16 = jnp.bfloat16


def _enc_body(x_ref, w0_ref, b0_ref, g0_ref, t0_ref, w1_ref, b1_ref, g1_ref,
              t1_ref, o_ref):
    h = jnp.dot(x_ref[...].astype(_bf16), w0_ref[...].astype(_bf16),
                preferred_element_type=_f32)
    h = (h + b0_ref[...]) * (g0_ref[...] * _BNS) + t0_ref[...]
    h = _elu(h)
    h = jnp.dot(h.astype(_bf16), w1_ref[...].astype(_bf16),
                preferred_element_type=_f32)
    h = (h + b1_ref[...]) * (g1_ref[...] * _BNS) + t1_ref[...]
    o_ref[...] = _elu(h)


def _row_spec(w):
    return pl.BlockSpec((BR, w), lambda i: (i, 0))


def _full_spec(shape):
    return pl.BlockSpec(shape, lambda i: (0,) * len(shape))


def _encoder(x, w0, b0, g0, t0, w1, b1, g1, t1):
    return pl.pallas_call(
        _enc_body,
        grid=(NB,),
        in_specs=[_row_spec(D_IN), _full_spec((D_IN, H0)), _full_spec((1, H0)),
                  _full_spec((1, H0)), _full_spec((1, H0)),
                  _full_spec((H0, H1)), _full_spec((1, H1)),
                  _full_spec((1, H1)), _full_spec((1, H1))],
        out_specs=_row_spec(H1),
        out_shape=jax.ShapeDtypeStruct((N, H1), _f32),
    )(x, w0, b0.reshape(1, H0), g0.reshape(1, H0), t0.reshape(1, H0),
      w1, b1.reshape(1, H1), g1.reshape(1, H1), t1.reshape(1, H1))


def _prep_body(z_ref, w_ref, cnt_ref, ylo_ref, yhi_ref):
    xw = jnp.dot(z_ref[...], w_ref[...], preferred_element_type=_f32)
    dinv = lax.rsqrt(cnt_ref[:, 0:1] + 1.0)
    y = xw * dinv
    ylo_ref[...] = y[:, :32]
    yhi_ref[...] = y[:, 32:]


def _gcn_prep(z, w, cnt):
    return pl.pallas_call(
        _prep_body,
        grid=(NB,),
        in_specs=[_row_spec(H1), _full_spec((H1, HID)), _row_spec(8)],
        out_specs=[_row_spec(32), _row_spec(32)],
        out_shape=[jax.ShapeDtypeStruct((N, 32), _f32)] * 2,
    )(z, w, cnt)


def _mid_body(plo_ref, phi_ref, cnt_ref, b_ref, a_ref, w_ref,
              ylo_ref, yhi_ref):
    a = a_ref[0, 0]
    dinv = lax.rsqrt(cnt_ref[:, 0:1] + 1.0)
    p = jnp.concatenate([plo_ref[...], phi_ref[...]], axis=1)
    h = p * dinv + b_ref[...]
    h = jnp.where(h >= 0, h, a * h)
    xw = jnp.dot(h, w_ref[...], preferred_element_type=_f32)
    y = xw * dinv
    ylo_ref[...] = y[:, :32]
    yhi_ref[...] = y[:, 32:]


def _gcn_mid(plo, phi, cnt, b, a, w):
    return pl.pallas_call(
        _mid_body,
        grid=(NB,),
        in_specs=[_row_spec(32), _row_spec(32), _row_spec(8),
                  _full_spec((1, HID)), _full_spec((1, 1)),
                  _full_spec((HID, PROJ))],
        out_specs=[_row_spec(32), _row_spec(32)],
        out_shape=[jax.ShapeDtypeStruct((N, 32), _f32)] * 2,
    )(plo, phi, cnt, b.reshape(1, HID), a.reshape(1, 1), w)


def _head_body(p1lo_ref, p1hi_ref, p2lo_ref, p2hi_ref, cnt1_ref, cnt2_ref,
               cb_ref, a_ref,
               f1w_ref, f1b_ref, f2w_ref, f2b_ref, aw_ref,
               dw_ref, db_ref, dg_ref, dt_ref,
               h1_ref, h2_ref, emb_ref, dz_ref):
    a = a_ref[0, 0]

    def fin(plo_ref, phi_ref, cnt_ref):
        dinv = lax.rsqrt(cnt_ref[:, 0:1] + 1.0)
        p = jnp.concatenate([plo_ref[...], phi_ref[...]], axis=1)
        h = p * dinv + cb_ref[...]
        return jnp.where(h >= 0, h, a * h)

    e1 = fin(p1lo_ref, p1hi_ref, cnt1_ref)
    e2 = fin(p2lo_ref, p2hi_ref, cnt2_ref)

    def proj(e):
        t = _elu(jnp.dot(e, f1w_ref[...], preferred_element_type=_f32)
                 + f1b_ref[...])
        return jnp.dot(t, f2w_ref[...], preferred_element_type=_f32) + f2b_ref[...]

    h1_ref[...] = proj(e1)
    h2_ref[...] = proj(e2)

    w1 = jnp.dot(e1, aw_ref[...], preferred_element_type=_f32)
    w2 = jnp.dot(e2, aw_ref[...], preferred_element_type=_f32)
    m = jnp.maximum(w1, w2)
    x1 = jnp.exp(w1 - m)
    x2 = jnp.exp(w2 - m)
    b1 = x1 / (x1 + x2)
    emb = b1 * e1 + (1.0 - b1) * e2
    emb_ref[...] = emb

    d = jnp.dot(emb, dw_ref[...], preferred_element_type=_f32)
    d = (d + db_ref[...]) * (dg_ref[...] * _BNS) + dt_ref[...]
    dz_ref[...] = 1.0 / (1.0 + jnp.exp(-d))


def _head(p1lo, p1hi, p2lo, p2hi, cnt1, cnt2, cb, a,
          f1w, f1b, f2w, f2b, aw, dw, db, dg, dt):
    return pl.pallas_call(
        _head_body,
        grid=(NB,),
        in_specs=[_row_spec(32), _row_spec(32), _row_spec(32), _row_spec(32),
                  _row_spec(8), _row_spec(8),
                  _full_spec((1, PROJ)), _full_spec((1, 1)),
                  _full_spec((PROJ, PROJ)), _full_spec((1, PROJ)),
                  _full_spec((PROJ, PROJ)), _full_spec((1, PROJ)),
                  _full_spec((PROJ, 1)),
                  _full_spec((PROJ, D_IN)), _full_spec((1, D_IN)),
                  _full_spec((1, D_IN)), _full_spec((1, D_IN))],
        out_specs=[_row_spec(PROJ), _row_spec(PROJ), _row_spec(PROJ),
                   _row_spec(D_IN)],
        out_shape=[jax.ShapeDtypeStruct((N, PROJ), _f32),
                   jax.ShapeDtypeStruct((N, PROJ), _f32),
                   jax.ShapeDtypeStruct((N, PROJ), _f32),
                   jax.ShapeDtypeStruct((N, D_IN), _f32)],
    )(p1lo, p1hi, p2lo, p2hi, cnt1, cnt2,
      cb.reshape(1, PROJ), a.reshape(1, 1),
      f1w, f1b.reshape(1, PROJ), f2w, f2b.reshape(1, PROJ), aw,
      dw, db.reshape(1, D_IN), dg.reshape(1, D_IN), dt.reshape(1, D_IN))


_MESH = plsc.VectorSubcoreMesh(core_axis_name="c", subcore_axis_name="s",
                               num_cores=2, num_subcores=NTILE)
_SC_PARAMS = pltpu.CompilerParams(use_tc_tiling_on_sc=False)


def _node_rows(sid, do_full, do_last):
    row0 = sid * RPT

    @pl.when(sid < NTILE - 1)
    def _():
        do_full(row0)

    @pl.when(sid == NTILE - 1)
    def _():
        do_last(row0)


@functools.partial(
    pl.kernel,
    out_type=[jax.ShapeDtypeStruct((N, 32), _f32),
              jax.ShapeDtypeStruct((N, 32), _f32)],
    mesh=_MESH,
    scratch_types=[pltpu.VMEM_SHARED((N, 32), _f32),
                   pltpu.VMEM((SEG, CHUNK), jnp.int32),
                   pltpu.VMEM((SEG, CHUNK), jnp.int32),
                   [pltpu.VMEM((CHUNK, 32), _f32) for _ in range(NBUF)],
                   [pltpu.SemaphoreType.DMA for _ in range(NBUF)],
                   [pltpu.SemaphoreType.DMA for _ in range(NBUF)]],
    compiler_params=_SC_PARAMS,
)
def _edge_pass(ylo, yhi, src2d, dst2d, plo, phi,
               acc, sidx, didx, rows, sems, sems2):
    cid = lax.axis_index("c")
    sid = lax.axis_index("s")

    cbase = (CPT - 1) * sid + jnp.minimum(sid, XTILES)
    has_extra = sid < XTILES

    def run(y_hbm, out_hbm):
        _node_rows(
            sid,
            lambda r0: pltpu.sync_copy(y_hbm.at[pl.ds(r0, RPT)],
                                       acc.at[pl.ds(r0, RPT)]),
            lambda r0: pltpu.sync_copy(y_hbm.at[pl.ds(r0, RLAST)],
                                       acc.at[pl.ds(r0, RLAST)]),
        )
        plsc.subcore_barrier()

        def start(c, b):
            pltpu.async_copy(y_hbm.at[sidx.at[c]], rows[b], sems[b])

        def wait(b):
            pltpu.make_async_copy(y_hbm.at[pl.ds(0, CHUNK)], rows[b],
                                  sems[b]).wait()

        def scat(c, b):
            pltpu.async_copy(rows[b], acc.at[didx.at[c]], sems2[b], add=True)

        def wait_scat(b):
            pltpu.make_async_copy(y_hbm.at[pl.ds(0, CHUNK)], rows[b],
                                  sems2[b]).wait()

        def seg_body(s, _):
            cb = cbase + s * SEG
            pltpu.sync_copy(src2d.at[pl.ds(cb, SEG)], sidx)
            pltpu.sync_copy(dst2d.at[pl.ds(cb, SEG)], didx)

            for b in range(NBUF):
                start(b, b)

            def group(g, _):
                c0 = g * NBUF
                for b in range(NBUF):
                    wait(b)
                    scat(c0 + b, b)
                    nxt = c0 + b + NBUF

                    @pl.when(nxt < SEG)
                    def _():
                        wait_scat(b)
                        start(nxt, b)
                return 0

            lax.fori_loop(0, SEG // NBUF, group, 0)

            for b in range(NBUF):
                wait_scat(b)
            return 0

        lax.fori_loop(0, NSEG, seg_body, 0)

        @pl.when(has_extra)
        def _():
            pltpu.sync_copy(src2d.at[pl.ds(cbase + CPT - 1, 1)],
                            sidx.at[pl.ds(0, 1)])
            pltpu.sync_copy(dst2d.at[pl.ds(cbase + CPT - 1, 1)],
                            didx.at[pl.ds(0, 1)])
            start(0, 0)
            wait(0)
            scat(0, 0)
            wait_scat(0)

        plsc.subcore_barrier()

        _node_rows(
            sid,
            lambda r0: pltpu.sync_copy(acc.at[pl.ds(r0, RPT)],
                                       out_hbm.at[pl.ds(r0, RPT)]),
            lambda r0: pltpu.sync_copy(acc.at[pl.ds(r0, RLAST)],
                                       out_hbm.at[pl.ds(r0, RLAST)]),
        )

    @pl.when(cid == 0)
    def _():
        run(ylo, plo)

    @pl.when(cid == 1)
    def _():
        run(yhi, phi)


@functools.partial(
    pl.kernel,
    out_type=[jax.ShapeDtypeStruct((N, 8), _f32),
              jax.ShapeDtypeStruct((N, 8), _f32)],
    mesh=_MESH,
    scratch_types=[pltpu.VMEM_SHARED((N, 8), _f32),
                   pltpu.VMEM((CPT, CHUNK), jnp.int32),
                   pltpu.VMEM((CHUNK, 8), _f32)],
    compiler_params=_SC_PARAMS,
)
def _degrees_sc(dst2d1, dst2d2, zeros, ones, cnt1, cnt2, acc, didx, ones_v):
    cid = lax.axis_index("c")
    sid = lax.axis_index("s")

    pltpu.sync_copy(ones, ones_v)

    cbase = (CPT - 1) * sid + jnp.minimum(sid, XTILES)
    has_extra = sid < XTILES

    def run(dst2d, out_hbm):
        pltpu.sync_copy(dst2d.at[pl.ds(cbase, CPT - 1)],
                        didx.at[pl.ds(0, CPT - 1)])

        @pl.when(has_extra)
        def _():
            pltpu.sync_copy(dst2d.at[pl.ds(cbase + CPT - 1, 1)],
                            didx.at[pl.ds(CPT - 1, 1)])

        _node_rows(
            sid,
            lambda r0: pltpu.sync_copy(zeros.at[pl.ds(r0, RPT)],
                                       acc.at[pl.ds(r0, RPT)]),
            lambda r0: pltpu.sync_copy(zeros.at[pl.ds(r0, RLAST)],
                                       acc.at[pl.ds(r0, RLAST)]),
        )
        plsc.subcore_barrier()

        def chunk(j, _):
            pltpu.sync_copy(ones_v, acc.at[didx.at[j]], add=True)
            return 0

        lax.fori_loop(0, CPT - 1, chunk, 0)

        @pl.when(has_extra)
        def _():
            pltpu.sync_copy(ones_v, acc.at[didx.at[CPT - 1]], add=True)

        plsc.subcore_barrier()

        _node_rows(
            sid,
            lambda r0: pltpu.sync_copy(acc.at[pl.ds(r0, RPT)],
                                       out_hbm.at[pl.ds(r0, RPT)]),
            lambda r0: pltpu.sync_copy(acc.at[pl.ds(r0, RLAST)],
                                       out_hbm.at[pl.ds(r0, RLAST)]),
        )

    @pl.when(cid == 0)
    def _():
        run(dst2d1, cnt1)

    @pl.when(cid == 1)
    def _():
        run(dst2d2, cnt2)


def kernel(x, edge_index_1, edge_index_2,
           enc_W0, enc_b0, enc_g0, enc_bt0,
           enc_W1, enc_b1, enc_g1, enc_bt1,
           conv_W0, conv_b0, conv_W1, conv_b1, prelu_a,
           fc1_W, fc1_b, fc2_W, fc2_b, att_W,
           dec_W, dec_b, dec_g, dec_bt):
    z = _encoder(x, enc_W0, enc_b0, enc_g0, enc_bt0,
                 enc_W1, enc_b1, enc_g1, enc_bt1)

    src1 = edge_index_1[0].reshape(NCH, CHUNK)
    dst1 = edge_index_1[1].reshape(NCH, CHUNK)
    src2 = edge_index_2[0].reshape(NCH, CHUNK)
    dst2 = edge_index_2[1].reshape(NCH, CHUNK)

    zeros8 = jnp.zeros((N, 8), _f32)
    ones8 = jnp.ones((CHUNK, 8), _f32)
    cnt1, cnt2 = _degrees_sc(dst1, dst2, zeros8, ones8)

    def branch(src, dst, cnt):
        ylo, yhi = _gcn_prep(z, conv_W0, cnt)
        plo, phi = _edge_pass(ylo, yhi, src, dst)
        yblo, ybhi = _gcn_mid(plo, phi, cnt, conv_b0, prelu_a, conv_W1)
        return _edge_pass(yblo, ybhi, src, dst)

    q1lo, q1hi = branch(src1, dst1, cnt1)
    q2lo, q2hi = branch(src2, dst2, cnt2)

    return _head(q1lo, q1hi, q2lo, q2hi, cnt1, cnt2, conv_b1, prelu_a,
                 fc1_W, fc1_b, fc2_W, fc2_b, att_W,
                 dec_W, dec_b, dec_g, dec_bt)
